# baseline (speedup 1.0000x reference)
import functools

import jax
import jax.numpy as jnp
from jax import lax
from jax.experimental import pallas as pl
from jax.experimental.pallas import tpu as pltpu
from jax.experimental.pallas import tpu_sc as plsc

BATCH = 16384
DIM = 16
NUM_CORES = 2
NUM_SUBCORES = 16
NUM_WORKERS = NUM_CORES * NUM_SUBCORES
BPW = BATCH // NUM_WORKERS
TILES = BPW // 16


def _mf_body(skill_hbm, attempt_hbm, p_hbm, q_hbm, pb_hbm, qb_hbm, out_hbm,
             sidx_v, aidx_v, prow_v, qrow_v, pb_v, qb_v, out_v, sem):
    wid = lax.axis_index("s") * NUM_CORES + lax.axis_index("c")
    base = wid * BPW

    pltpu.sync_copy(skill_hbm.at[pl.ds(base, BPW)], sidx_v)
    pltpu.sync_copy(attempt_hbm.at[pl.ds(base, BPW)], aidx_v)

    c1 = pltpu.async_copy(p_hbm.at[sidx_v], prow_v, sem)
    c2 = pltpu.async_copy(q_hbm.at[aidx_v], qrow_v, sem)
    c3 = pltpu.async_copy(pb_hbm.at[sidx_v], pb_v, sem)
    c4 = pltpu.async_copy(qb_hbm.at[aidx_v], qb_v, sem)
    c1.wait()
    c2.wait()
    c3.wait()
    c4.wait()

    def tile_body(t, carry):
        row_ids = t * 16 + lax.iota(jnp.int32, 16)
        acc = pb_v[pl.ds(t * 16, 16)] + qb_v[pl.ds(t * 16, 16)]
        for d in range(DIM):
            dcol = jnp.full((16,), d, jnp.int32)
            p = plsc.load_gather(prow_v, [row_ids, dcol])
            q = plsc.load_gather(qrow_v, [row_ids, dcol])
            acc = acc + p * q
        out_v[pl.ds(t * 16, 16)] = acc
        return carry

    lax.fori_loop(0, TILES, tile_body, 0)

    pltpu.sync_copy(out_v, out_hbm.at[pl.ds(base, BPW)])


@jax.jit
def _mf(skill_sequence, attempt_sequence, P, Q, P_bias, Q_bias):
    mesh = plsc.VectorSubcoreMesh(core_axis_name="c", subcore_axis_name="s")
    run = functools.partial(
        pl.kernel,
        out_type=jax.ShapeDtypeStruct((BATCH,), jnp.float32),
        mesh=mesh,
        compiler_params=pltpu.CompilerParams(
            needs_layout_passes=False,
            use_tc_tiling_on_sc=False,
        ),
        scratch_types=[
            pltpu.VMEM((BPW,), jnp.int32),
            pltpu.VMEM((BPW,), jnp.int32),
            pltpu.VMEM((BPW, DIM), jnp.float32),
            pltpu.VMEM((BPW, DIM), jnp.float32),
            pltpu.VMEM((BPW,), jnp.float32),
            pltpu.VMEM((BPW,), jnp.float32),
            pltpu.VMEM((BPW,), jnp.float32),
            pltpu.SemaphoreType.DMA,
        ],
    )(_mf_body)
    return run(skill_sequence, attempt_sequence, P, Q, P_bias, Q_bias)


def kernel(skill_sequence, attempt_sequence, P, Q, P_bias, Q_bias):
    out = _mf(skill_sequence.astype(jnp.int32),
              attempt_sequence.astype(jnp.int32),
              P, Q, P_bias.reshape(-1), Q_bias.reshape(-1))
    return out.reshape(BATCH, 1)
